# baseline (device time: 14405 ns/iter reference)
import jax
import jax.numpy as jnp
from jax import lax
from jax.experimental import pallas as pl
from jax.experimental.pallas import tpu as pltpu

N_DEV = 8
C = 8
OV = 8


def kernel(x):
    m, n = x.shape
    mc = m // C

    def in_rows(c):
        lo = 0 if c == 0 else c * mc - OV
        hi = m if c == C - 1 else c * mc + mc + OV
        return lo, hi - lo

    def body(x_hbm, o_hbm, halo_ref, xbuf, obuf, s_ref,
             in_sems, out_sems, send_sems, recv_sems):
        my = lax.axis_index("i")
        has_left = my > 0
        has_right = my < N_DEV - 1

        barrier = pltpu.get_barrier_semaphore()

        @pl.when(has_left)
        def _():
            pl.semaphore_signal(
                barrier, inc=1, device_id=(my - 1,),
                device_id_type=pl.DeviceIdType.MESH,
            )

        @pl.when(jnp.logical_not(has_left))
        def _():
            pl.semaphore_signal(barrier, inc=1)

        @pl.when(has_right)
        def _():
            pl.semaphore_signal(
                barrier, inc=1, device_id=(my + 1,),
                device_id_type=pl.DeviceIdType.MESH,
            )

        @pl.when(jnp.logical_not(has_right))
        def _():
            pl.semaphore_signal(barrier, inc=1)

        pl.semaphore_wait(barrier, 2)

        send_left = pltpu.make_async_remote_copy(
            src_ref=x_hbm.at[pl.ds(0, 1)],
            dst_ref=halo_ref.at[1],
            send_sem=send_sems.at[0],
            recv_sem=recv_sems.at[1],
            device_id=(my - 1,),
            device_id_type=pl.DeviceIdType.MESH,
        )
        send_right = pltpu.make_async_remote_copy(
            src_ref=x_hbm.at[pl.ds(m - 1, 1)],
            dst_ref=halo_ref.at[0],
            send_sem=send_sems.at[1],
            recv_sem=recv_sems.at[0],
            device_id=(my + 1,),
            device_id_type=pl.DeviceIdType.MESH,
        )
        recv_from_left = pltpu.make_async_remote_copy(
            src_ref=x_hbm.at[pl.ds(0, 1)],
            dst_ref=halo_ref.at[0],
            send_sem=send_sems.at[0],
            recv_sem=recv_sems.at[0],
            device_id=(my,),
            device_id_type=pl.DeviceIdType.MESH,
        )
        recv_from_right = pltpu.make_async_remote_copy(
            src_ref=x_hbm.at[pl.ds(0, 1)],
            dst_ref=halo_ref.at[1],
            send_sem=send_sems.at[0],
            recv_sem=recv_sems.at[1],
            device_id=(my,),
            device_id_type=pl.DeviceIdType.MESH,
        )

        @pl.when(has_left)
        def _():
            send_left.start()

        @pl.when(has_right)
        def _():
            send_right.start()

        order = list(range(1, C - 1)) + [0, C - 1]

        in_pending = {}

        def start_in(c, slot):
            lo, cnt = in_rows(c)
            cp = pltpu.make_async_copy(
                x_hbm.at[pl.ds(lo, cnt)],
                xbuf.at[slot, pl.ds(0, cnt)],
                in_sems.at[slot],
            )
            cp.start()
            in_pending[c] = cp

        start_in(order[0], 0)
        start_in(order[1], 1)

        out_pending = [None, None]

        def stencil_chunk(k, c):
            islot = k % 3
            oslot = k % 2
            if k + 2 < C:
                start_in(order[k + 2], (k + 2) % 3)
            in_pending[c].wait()
            a = c * mc
            off = 0 if c == 0 else OV
            j0 = 1 if c == 0 else 0
            j1 = mc - 1 if c == C - 1 else mc
            cnt = j1 + 1 - j0
            s_ref[pl.ds(j0, cnt), :] = (
                xbuf[islot, pl.ds(off - 1 + j0, cnt), :]
                + xbuf[islot, pl.ds(off + j0, cnt), :]
            )
            if out_pending[oslot] is not None:
                out_pending[oslot].wait()
            obuf[oslot, pl.ds(j0, j1 - j0), :] = (
                0.25 * (s_ref[pl.ds(j0, j1 - j0), :]
                        + s_ref[pl.ds(j0 + 1, j1 - j0), :])
            ).astype(obuf.dtype)

        def flush_chunk(k, c):
            oslot = k % 2
            cp = pltpu.make_async_copy(
                obuf.at[oslot],
                o_hbm.at[pl.ds(c * mc, mc)],
                out_sems.at[oslot],
            )
            cp.start()
            out_pending[oslot] = cp

        for k, c in enumerate(order[: C - 2]):
            stencil_chunk(k, c)
            flush_chunk(k, c)

        k0, kL = C - 2, C - 1
        stencil_chunk(k0, 0)
        stencil_chunk(kL, C - 1)
        oslot0, islot0 = k0 % 2, k0 % 3
        oslotL, islotL = kL % 2, kL % 3

        @pl.when(has_left)
        def _():
            recv_from_left.wait_recv()
            obuf[oslot0, pl.ds(0, 1), :] = (
                0.25 * halo_ref[0]
                + 0.5 * xbuf[islot0, pl.ds(0, 1), :]
                + 0.25 * xbuf[islot0, pl.ds(1, 1), :]
            ).astype(obuf.dtype)

        @pl.when(jnp.logical_not(has_left))
        def _():
            obuf[oslot0, pl.ds(0, 1), :] = xbuf[
                islot0, pl.ds(0, 1), :
            ].astype(obuf.dtype)

        flush_chunk(k0, 0)

        @pl.when(has_right)
        def _():
            recv_from_right.wait_recv()
            obuf[oslotL, pl.ds(mc - 1, 1), :] = (
                0.25 * xbuf[islotL, pl.ds(OV + mc - 2, 1), :]
                + 0.5 * xbuf[islotL, pl.ds(OV + mc - 1, 1), :]
                + 0.25 * halo_ref[1]
            ).astype(obuf.dtype)

        @pl.when(jnp.logical_not(has_right))
        def _():
            obuf[oslotL, pl.ds(mc - 1, 1), :] = xbuf[
                islotL, pl.ds(OV + mc - 1, 1), :
            ].astype(obuf.dtype)

        flush_chunk(kL, C - 1)

        out_pending[0].wait()
        out_pending[1].wait()

        @pl.when(has_left)
        def _():
            send_left.wait_send()

        @pl.when(has_right)
        def _():
            send_right.wait_send()

    return pl.pallas_call(
        body,
        out_shape=jax.ShapeDtypeStruct((m, n), jnp.bfloat16),
        in_specs=[pl.BlockSpec(memory_space=pltpu.MemorySpace.HBM)],
        out_specs=pl.BlockSpec(memory_space=pltpu.MemorySpace.HBM),
        scratch_shapes=[
            pltpu.VMEM((2, 1, n), x.dtype),
            pltpu.VMEM((3, mc + 2 * OV, n), x.dtype),
            pltpu.VMEM((2, mc, n), jnp.bfloat16),
            pltpu.VMEM((mc + 1, n), x.dtype),
            pltpu.SemaphoreType.DMA((3,)),
            pltpu.SemaphoreType.DMA((2,)),
            pltpu.SemaphoreType.DMA((2,)),
            pltpu.SemaphoreType.DMA((2,)),
        ],
        compiler_params=pltpu.CompilerParams(collective_id=0),
    )(x)


# device time: 14051 ns/iter; 1.0252x vs baseline; 1.0252x over previous
import jax
import jax.numpy as jnp
from jax import lax
from jax.experimental import pallas as pl
from jax.experimental.pallas import tpu as pltpu

N_DEV = 8
C = 8
OV = 8


def kernel(x):
    m, n = x.shape
    mc = m // C

    def in_rows(c):
        lo = 0 if c == 0 else c * mc - OV
        hi = m if c == C - 1 else c * mc + mc + OV
        return lo, hi - lo

    def body(x_hbm, o_hbm, halo_ref, xbuf, obuf, s_ref,
             in_sems, out_sems, send_sems, recv_sems):
        my = lax.axis_index("i")
        has_left = my > 0
        has_right = my < N_DEV - 1

        barrier = pltpu.get_barrier_semaphore()

        @pl.when(has_left)
        def _():
            pl.semaphore_signal(
                barrier, inc=1, device_id=(my - 1,),
                device_id_type=pl.DeviceIdType.MESH,
            )

        @pl.when(jnp.logical_not(has_left))
        def _():
            pl.semaphore_signal(barrier, inc=1)

        @pl.when(has_right)
        def _():
            pl.semaphore_signal(
                barrier, inc=1, device_id=(my + 1,),
                device_id_type=pl.DeviceIdType.MESH,
            )

        @pl.when(jnp.logical_not(has_right))
        def _():
            pl.semaphore_signal(barrier, inc=1)


        send_left = pltpu.make_async_remote_copy(
            src_ref=x_hbm.at[pl.ds(0, 1)],
            dst_ref=halo_ref.at[1],
            send_sem=send_sems.at[0],
            recv_sem=recv_sems.at[1],
            device_id=(my - 1,),
            device_id_type=pl.DeviceIdType.MESH,
        )
        send_right = pltpu.make_async_remote_copy(
            src_ref=x_hbm.at[pl.ds(m - 1, 1)],
            dst_ref=halo_ref.at[0],
            send_sem=send_sems.at[1],
            recv_sem=recv_sems.at[0],
            device_id=(my + 1,),
            device_id_type=pl.DeviceIdType.MESH,
        )
        recv_from_left = pltpu.make_async_remote_copy(
            src_ref=x_hbm.at[pl.ds(0, 1)],
            dst_ref=halo_ref.at[0],
            send_sem=send_sems.at[0],
            recv_sem=recv_sems.at[0],
            device_id=(my,),
            device_id_type=pl.DeviceIdType.MESH,
        )
        recv_from_right = pltpu.make_async_remote_copy(
            src_ref=x_hbm.at[pl.ds(0, 1)],
            dst_ref=halo_ref.at[1],
            send_sem=send_sems.at[0],
            recv_sem=recv_sems.at[1],
            device_id=(my,),
            device_id_type=pl.DeviceIdType.MESH,
        )

        order = list(range(1, C - 1)) + [0, C - 1]

        in_pending = {}

        def start_in(c, slot):
            lo, cnt = in_rows(c)
            cp = pltpu.make_async_copy(
                x_hbm.at[pl.ds(lo, cnt)],
                xbuf.at[slot, pl.ds(0, cnt)],
                in_sems.at[slot],
            )
            cp.start()
            in_pending[c] = cp

        start_in(order[0], 0)
        start_in(order[1], 1)

        out_pending = [None, None]

        def stencil_chunk(k, c):
            islot = k % 3
            oslot = k % 2
            if k + 2 < C:
                start_in(order[k + 2], (k + 2) % 3)
            in_pending[c].wait()
            a = c * mc
            off = 0 if c == 0 else OV
            j0 = 1 if c == 0 else 0
            j1 = mc - 1 if c == C - 1 else mc
            cnt = j1 + 1 - j0
            s_ref[pl.ds(j0, cnt), :] = (
                xbuf[islot, pl.ds(off - 1 + j0, cnt), :]
                + xbuf[islot, pl.ds(off + j0, cnt), :]
            )
            if out_pending[oslot] is not None:
                out_pending[oslot].wait()
            obuf[oslot, pl.ds(j0, j1 - j0), :] = (
                0.25 * (s_ref[pl.ds(j0, j1 - j0), :]
                        + s_ref[pl.ds(j0 + 1, j1 - j0), :])
            ).astype(obuf.dtype)

        def flush_chunk(k, c):
            oslot = k % 2
            cp = pltpu.make_async_copy(
                obuf.at[oslot],
                o_hbm.at[pl.ds(c * mc, mc)],
                out_sems.at[oslot],
            )
            cp.start()
            out_pending[oslot] = cp

        for k, c in enumerate(order[: C - 2]):
            stencil_chunk(k, c)
            flush_chunk(k, c)

        pl.semaphore_wait(barrier, 2)

        @pl.when(has_left)
        def _():
            send_left.start()

        @pl.when(has_right)
        def _():
            send_right.start()

        k0, kL = C - 2, C - 1
        stencil_chunk(k0, 0)
        stencil_chunk(kL, C - 1)
        oslot0, islot0 = k0 % 2, k0 % 3
        oslotL, islotL = kL % 2, kL % 3

        @pl.when(has_left)
        def _():
            recv_from_left.wait_recv()
            obuf[oslot0, pl.ds(0, 1), :] = (
                0.25 * halo_ref[0]
                + 0.5 * xbuf[islot0, pl.ds(0, 1), :]
                + 0.25 * xbuf[islot0, pl.ds(1, 1), :]
            ).astype(obuf.dtype)

        @pl.when(jnp.logical_not(has_left))
        def _():
            obuf[oslot0, pl.ds(0, 1), :] = xbuf[
                islot0, pl.ds(0, 1), :
            ].astype(obuf.dtype)

        flush_chunk(k0, 0)

        @pl.when(has_right)
        def _():
            recv_from_right.wait_recv()
            obuf[oslotL, pl.ds(mc - 1, 1), :] = (
                0.25 * xbuf[islotL, pl.ds(OV + mc - 2, 1), :]
                + 0.5 * xbuf[islotL, pl.ds(OV + mc - 1, 1), :]
                + 0.25 * halo_ref[1]
            ).astype(obuf.dtype)

        @pl.when(jnp.logical_not(has_right))
        def _():
            obuf[oslotL, pl.ds(mc - 1, 1), :] = xbuf[
                islotL, pl.ds(OV + mc - 1, 1), :
            ].astype(obuf.dtype)

        flush_chunk(kL, C - 1)

        out_pending[0].wait()
        out_pending[1].wait()

        @pl.when(has_left)
        def _():
            send_left.wait_send()

        @pl.when(has_right)
        def _():
            send_right.wait_send()

    return pl.pallas_call(
        body,
        out_shape=jax.ShapeDtypeStruct((m, n), jnp.bfloat16),
        in_specs=[pl.BlockSpec(memory_space=pltpu.MemorySpace.HBM)],
        out_specs=pl.BlockSpec(memory_space=pltpu.MemorySpace.HBM),
        scratch_shapes=[
            pltpu.VMEM((2, 1, n), x.dtype),
            pltpu.VMEM((3, mc + 2 * OV, n), x.dtype),
            pltpu.VMEM((2, mc, n), jnp.bfloat16),
            pltpu.VMEM((mc + 1, n), x.dtype),
            pltpu.SemaphoreType.DMA((3,)),
            pltpu.SemaphoreType.DMA((2,)),
            pltpu.SemaphoreType.DMA((2,)),
            pltpu.SemaphoreType.DMA((2,)),
        ],
        compiler_params=pltpu.CompilerParams(collective_id=0),
    )(x)
